# Initial kernel scaffold; baseline (speedup 1.0000x reference)
import functools

import jax
import jax.numpy as jnp
from jax import lax
from jax.experimental import pallas as pl
from jax.experimental.pallas import tpu as pltpu
from jax.experimental.pallas import tpu_sc as plsc

NC = 2
NS = 16
NW = NC * NS
L = 128
CHUNK_BLKS = 10
CHUNK = CHUNK_BLKS * L


@functools.lru_cache(maxsize=None)
def _make_gather(n_ctx, n_q, vocab, d):
    ctx_blocks = n_ctx // (NW * L)
    q_blocks = n_q // (NW * L)
    assert ctx_blocks * NW * L == n_ctx and q_blocks * NW * L == n_q
    assert ctx_blocks % CHUNK_BLKS == 0 and q_blocks % CHUNK_BLKS == 0

    mesh = plsc.VectorSubcoreMesh(core_axis_name="c", subcore_axis_name="s")

    @functools.partial(
        pl.kernel,
        mesh=mesh,
        out_type=[
            jax.ShapeDtypeStruct((n_ctx, d), jnp.float32),
            jax.ShapeDtypeStruct((n_q, d), jnp.float32),
        ],
        scratch_types=[
            pltpu.VMEM((CHUNK_BLKS, L), jnp.int32),
            pltpu.VMEM((CHUNK, d), jnp.float32),
            pltpu.SemaphoreType.DMA,
        ],
    )
    def gather_kernel(ctx_idx, q_idx, table, ctx_out, q_out, idx_v, rows_v, sem):
        wid = lax.axis_index("s") * NC + lax.axis_index("c")

        def phase(idx_hbm, out_hbm, blocks_per_worker):
            blk_base = wid * blocks_per_worker
            row_base = blk_base * L

            def chunk_body(c, carry):
                pltpu.sync_copy(
                    idx_hbm.at[pl.ds(blk_base + c * CHUNK_BLKS, CHUNK_BLKS)],
                    idx_v,
                )
                copies = [
                    pltpu.async_copy(
                        table.at[idx_v.at[j]],
                        rows_v.at[pl.ds(j * L, L)],
                        sem,
                    )
                    for j in range(CHUNK_BLKS)
                ]
                for cp in copies:
                    cp.wait()
                pltpu.sync_copy(
                    rows_v,
                    out_hbm.at[pl.ds(row_base + c * CHUNK, CHUNK)],
                )
                return carry

            lax.fori_loop(0, blocks_per_worker // CHUNK_BLKS, chunk_body, 0)

        phase(ctx_idx, ctx_out, ctx_blocks)
        phase(q_idx, q_out, q_blocks)

    return gather_kernel


def kernel(input_context, input_question, word_embedding_weight):
    batch, ctx_len = input_context.shape
    _, q_len = input_question.shape
    vocab, d = word_embedding_weight.shape
    n_ctx = batch * ctx_len
    n_q = batch * q_len

    ctx_idx = input_context.astype(jnp.int32).reshape(n_ctx // L, L)
    q_idx = input_question.astype(jnp.int32).reshape(n_q // L, L)

    ctx_flat, q_flat = _make_gather(n_ctx, n_q, vocab, d)(
        ctx_idx, q_idx, word_embedding_weight
    )
    return (
        ctx_flat.reshape(batch, ctx_len, d),
        q_flat.reshape(batch, q_len, d),
    )


# speedup vs baseline: 1.5640x; 1.5640x over previous
import functools

import jax
import jax.numpy as jnp
from jax import lax
from jax.experimental import pallas as pl
from jax.experimental.pallas import tpu as pltpu
from jax.experimental.pallas import tpu_sc as plsc

NC = 2
NS = 16
NW = NC * NS
L = 128
CHUNK_BLKS = 10
CHUNK = CHUNK_BLKS * L


@functools.lru_cache(maxsize=None)
def _make_gather(n_ctx, n_q, vocab, d):
    ctx_blocks = n_ctx // (NW * L)
    q_blocks = n_q // (NW * L)
    assert ctx_blocks * NW * L == n_ctx and q_blocks * NW * L == n_q
    assert ctx_blocks % CHUNK_BLKS == 0 and q_blocks % CHUNK_BLKS == 0

    mesh = plsc.VectorSubcoreMesh(core_axis_name="c", subcore_axis_name="s")

    @functools.partial(
        pl.kernel,
        mesh=mesh,
        out_type=[
            jax.ShapeDtypeStruct((n_ctx, d), jnp.float32),
            jax.ShapeDtypeStruct((n_q, d), jnp.float32),
        ],
        scratch_types=[
            pltpu.VMEM((CHUNK,), jnp.int32),
            pltpu.VMEM((CHUNK, d), jnp.float32),
            pltpu.SemaphoreType.DMA,
        ],
        compiler_params=pltpu.CompilerParams(use_tc_tiling_on_sc=False),
    )
    def gather_kernel(ctx_idx, q_idx, table, ctx_out, q_out, idx_v, rows_v, sem):
        wid = lax.axis_index("s") * NC + lax.axis_index("c")

        def phase(idx_hbm, out_hbm, blocks_per_worker):
            row_base = wid * blocks_per_worker * L

            def chunk_body(c, carry):
                pltpu.sync_copy(
                    idx_hbm.at[pl.ds(row_base + c * CHUNK, CHUNK)],
                    idx_v,
                )
                copies = [
                    pltpu.async_copy(
                        table.at[idx_v.at[pl.ds(j * L, L)]],
                        rows_v.at[pl.ds(j * L, L)],
                        sem,
                    )
                    for j in range(CHUNK_BLKS)
                ]
                for cp in copies:
                    cp.wait()
                pltpu.sync_copy(
                    rows_v,
                    out_hbm.at[pl.ds(row_base + c * CHUNK, CHUNK)],
                )
                return carry

            lax.fori_loop(0, blocks_per_worker // CHUNK_BLKS, chunk_body, 0)

        phase(ctx_idx, ctx_out, ctx_blocks)
        phase(q_idx, q_out, q_blocks)

    return gather_kernel


def kernel(input_context, input_question, word_embedding_weight):
    batch, ctx_len = input_context.shape
    _, q_len = input_question.shape
    vocab, d = word_embedding_weight.shape
    n_ctx = batch * ctx_len
    n_q = batch * q_len

    ctx_idx = input_context.astype(jnp.int32).reshape(n_ctx)
    q_idx = input_question.astype(jnp.int32).reshape(n_q)

    ctx_flat, q_flat = _make_gather(n_ctx, n_q, vocab, d)(
        ctx_idx, q_idx, word_embedding_weight
    )
    return (
        ctx_flat.reshape(batch, ctx_len, d),
        q_flat.reshape(batch, q_len, d),
    )
